# baseline (speedup 1.0000x reference)
import functools

import jax
import jax.numpy as jnp
from jax import lax
from jax.experimental import pallas as pl
from jax.experimental.pallas import tpu as pltpu
from jax.experimental.pallas import tpu_sc as plsc

N = 16
IN_F = 128
HID = 128
MSG = 64
OUT_F = 64
CAT = HID + MSG
MAXM = N * 5
QPAD = MAXM + 16


def _enc_body(x_ref, wenc_ref, benc_ref, ei_ref, wns_ref, wmsg_ref,
              preds_ref, adj_ref, wnst_ref, wmsgt_ref):
    preds_ref[:] = lax.dot_general(
        x_ref[:], wenc_ref[:], (((1,), (1,)), ((), ())),
        preferred_element_type=jnp.float32) + benc_ref[:]
    src = ei_ref[0:1, :]
    dst = ei_ref[1:2, :]
    ids = lax.broadcasted_iota(jnp.int32, (N, ei_ref.shape[1]), 0)
    S = (ids == src).astype(jnp.float32)
    D = (ids == dst).astype(jnp.float32)
    a = lax.dot_general(S, D, (((1,), (1,)), ((), ())),
                        preferred_element_type=jnp.float32)
    a = a + lax.dot_general(D, S, (((1,), (1,)), ((), ())),
                            preferred_element_type=jnp.float32)
    adj_ref[:] = (a > 0).astype(jnp.float32)
    wnst_ref[:] = wns_ref[:].T
    wmsgt_ref[:] = wmsg_ref[:].T


def _walk_body(preds0_hbm, adj_hbm, wns_hbm, wmsg_hbm, bns_hbm, bmsg_hbm,
               fmsg_hbm, out_hbm,
               preds_v, adj_v, wns_v, wmsg_v, bns_v, bmsg_v,
               qnode_v, qowner_v, msglog_v, h_v, stage_v, part_v, sh_v, sem):
    c = lax.axis_index("c")
    s = lax.axis_index("s")
    rep = c * 8 + s // 2
    part = s % 2
    partner = s - 2 * part + 1

    pltpu.sync_copy(preds0_hbm, preds_v.at[pl.ds(0, N * HID)])
    pltpu.sync_copy(adj_hbm, adj_v)
    pltpu.sync_copy(wns_hbm, wns_v)
    pltpu.sync_copy(wmsg_hbm, wmsg_v)
    pltpu.sync_copy(bns_hbm, bns_v)
    pltpu.sync_copy(bmsg_hbm, bmsg_v)
    pltpu.sync_copy(fmsg_hbm, msglog_v.at[pl.ds(MAXM * MSG, MSG)])

    iotaf = lax.iota(jnp.int32, 16).astype(jnp.float32)
    qnode_v[pl.ds(0, 16)] = jnp.full((16,), rep, jnp.int32)
    qowner_v[pl.ds(0, 16)] = jnp.full((16,), MAXM, jnp.int32)

    hoff = part * (HID // 2)
    pscale = jnp.where(part == 0, 1.0, 0.0)

    def step(t, carry):
        head, tail = carry
        act = head < tail
        node = jnp.clip(qnode_v[pl.ds(head, 16)][0], 0, N - 1)
        owner = jnp.clip(qowner_v[pl.ds(head, 16)][0], 0, MAXM)
        msgch = [msglog_v[pl.ds(owner * MSG + j * 16, 16)]
                 for j in range(MSG // 16)]

        pbase = node * HID

        def dot_ns(kc, accs):
            hv = preds_v[pl.ds(pbase + kc * 16, 16)]
            for l in range(16):
                hk = hv[l]
                base = (kc * 16 + l) * HID + hoff
                accs = tuple(accs[j] + hk * wns_v[pl.ds(base + j * 16, 16)]
                             for j in range(4))
            return accs

        accs = lax.fori_loop(
            0, HID // 16, dot_ns,
            tuple(bns_v[pl.ds(hoff + j * 16, 16)] for j in range(4)))
        for mc in range(MSG // 16):
            hv = msgch[mc]
            for l in range(16):
                hk = hv[l]
                base = (HID + mc * 16 + l) * HID + hoff
                accs = tuple(accs[j] + hk * wns_v[pl.ds(base + j * 16, 16)]
                             for j in range(4))
        rel = [jnp.maximum(a, 0.0) for a in accs]

        maccs = [bmsg_v[pl.ds(j * 16, 16)] * pscale for j in range(4)]
        for jl in range(4):
            hv = rel[jl]
            for l in range(16):
                hk = hv[l]
                base = (hoff + jl * 16 + l) * MSG
                maccs = [maccs[j] + hk * wmsg_v[pl.ds(base + j * 16, 16)]
                         for j in range(4)]
        for m in range(2):
            hv = msgch[m] * pscale + msgch[2 + m] * (1.0 - pscale)
            for l in range(16):
                hk = hv[l]
                base = (HID + part * 32 + m * 16 + l) * MSG
                maccs = [maccs[j] + hk * wmsg_v[pl.ds(base + j * 16, 16)]
                         for j in range(4)]

        for j in range(4):
            stage_v[pl.ds(j * 16, 16)] = rel[j]
            stage_v[pl.ds(64 + j * 16, 16)] = maccs[j]
        par = t % 2
        cp = pltpu.async_copy(stage_v, sh_v.at[par, s], sem)

        arow = adj_v[pl.ds(node * 16, 16)]
        aeff = arow * jnp.where(act, 1.0, 0.0)
        qf = jnp.zeros((16,), jnp.float32)
        pf = jnp.float32(0.0)
        for l in range(16):
            al = aeff[l]
            oh = jnp.where(iotaf == pf, 1.0, 0.0)
            qf = qf + (al * float(l)) * oh
            pf = pf + al
        base = jnp.minimum(tail, MAXM)
        qnode_v[pl.ds(base, 16)] = qf.astype(jnp.int32)
        qowner_v[pl.ds(base, 16)] = jnp.full((16,), t, jnp.int32)

        cp.wait()
        plsc.subcore_barrier()
        pltpu.sync_copy(sh_v.at[par, partner], part_v)

        node_eff = jnp.where(act, node, N)
        poff = (HID // 2) - hoff
        for j in range(4):
            preds_v[pl.ds(node_eff * HID + hoff + j * 16, 16)] = rel[j]
            preds_v[pl.ds(node_eff * HID + poff + j * 16, 16)] = \
                part_v[pl.ds(j * 16, 16)]
        for j in range(4):
            msglog_v[pl.ds(t * MSG + j * 16, 16)] = (
                maccs[j] + part_v[pl.ds(64 + j * 16, 16)])

        return (head + jnp.where(act, 1, 0),
                tail + pf.astype(jnp.int32))

    lax.fori_loop(0, MAXM, step, (jnp.int32(0), jnp.int32(1)))

    @pl.when(part == 0)
    def _():
        pltpu.sync_copy(preds_v.at[pl.ds(rep * HID, HID)],
                        out_hbm.at[pl.ds(rep * HID, HID)])


def _dec_body(p_ref, wdec_ref, bdec_ref, o_ref):
    logits = lax.dot_general(
        p_ref[:], wdec_ref[:], (((1,), (1,)), ((), ())),
        preferred_element_type=jnp.float32) + bdec_ref[:]
    z = logits - jnp.max(logits, axis=-1, keepdims=True)
    o_ref[:] = z - jnp.log(jnp.sum(jnp.exp(z), axis=-1, keepdims=True))


def kernel(x, edge_index, W_enc, b_enc, W_ns, b_ns, W_msg, b_msg,
           W_dec, b_dec, first_message):
    ei = edge_index.astype(jnp.int32)
    ei = jnp.concatenate([ei, ei, ei, ei], axis=0)

    preds0, adj, wnst, wmsgt = pl.pallas_call(
        _enc_body,
        out_shape=[
            jax.ShapeDtypeStruct((N, HID), jnp.float32),
            jax.ShapeDtypeStruct((N, N), jnp.float32),
            jax.ShapeDtypeStruct((CAT, HID), jnp.float32),
            jax.ShapeDtypeStruct((CAT, MSG), jnp.float32),
        ],
    )(x, W_enc, b_enc.reshape(1, HID), ei, W_ns, W_msg)

    mesh = plsc.VectorSubcoreMesh(core_axis_name="c", subcore_axis_name="s")
    walk = pl.kernel(
        _walk_body, mesh=mesh,
        out_type=jax.ShapeDtypeStruct((N * HID,), jnp.float32),
        scratch_types=[
            pltpu.VMEM(((N + 1) * HID,), jnp.float32),
            pltpu.VMEM((N * N,), jnp.float32),
            pltpu.VMEM((CAT * HID,), jnp.float32),
            pltpu.VMEM((CAT * MSG,), jnp.float32),
            pltpu.VMEM((HID,), jnp.float32),
            pltpu.VMEM((MSG,), jnp.float32),
            pltpu.VMEM((QPAD,), jnp.int32),
            pltpu.VMEM((QPAD,), jnp.int32),
            pltpu.VMEM(((MAXM + 1) * MSG,), jnp.float32),
            pltpu.VMEM((HID,), jnp.float32),
            pltpu.VMEM((HID,), jnp.float32),
            pltpu.VMEM_SHARED((2, 16, HID), jnp.float32),
            pltpu.SemaphoreType.DMA,
        ],
    )
    finalp = walk(preds0.reshape(-1), adj.reshape(-1), wnst.reshape(-1),
                  wmsgt.reshape(-1), b_ns, b_msg, first_message.reshape(-1))

    out = pl.pallas_call(
        _dec_body,
        out_shape=jax.ShapeDtypeStruct((N, OUT_F), jnp.float32),
    )(finalp.reshape(N, HID), W_dec, b_dec.reshape(1, OUT_F))
    return (out, 0)


# speedup vs baseline: 1.4300x; 1.4300x over previous
import functools

import jax
import jax.numpy as jnp
from jax import lax
from jax.experimental import pallas as pl
from jax.experimental.pallas import tpu as pltpu
from jax.experimental.pallas import tpu_sc as plsc

N = 16
IN_F = 128
HID = 128
MSG = 64
OUT_F = 64
CAT = HID + MSG
MAXM = N * 5
QPAD = MAXM + 16


def _enc_body(x_ref, wenc_ref, benc_ref, ei_ref, wns_ref, wmsg_ref,
              preds_ref, adj_ref, wnst_ref, wmsgt_ref):
    preds_ref[:] = lax.dot_general(
        x_ref[:], wenc_ref[:], (((1,), (1,)), ((), ())),
        preferred_element_type=jnp.float32) + benc_ref[:]
    src = ei_ref[0:1, :]
    dst = ei_ref[1:2, :]
    ids = lax.broadcasted_iota(jnp.int32, (N, ei_ref.shape[1]), 0)
    S = (ids == src).astype(jnp.float32)
    D = (ids == dst).astype(jnp.float32)
    a = lax.dot_general(S, D, (((1,), (1,)), ((), ())),
                        preferred_element_type=jnp.float32)
    a = a + lax.dot_general(D, S, (((1,), (1,)), ((), ())),
                            preferred_element_type=jnp.float32)
    adj_ref[:] = (a > 0).astype(jnp.float32)
    wnst_ref[:] = wns_ref[:].T
    wmsgt_ref[:] = wmsg_ref[:].T


def _walk_body(preds0_hbm, adj_hbm, wns_hbm, wmsg_hbm, bns_hbm, bmsg_hbm,
               fmsg_hbm, out_hbm,
               preds_v, adj_v, wns_v, wmsg_v, bns_v, bmsg_v,
               qnode_v, qowner_v, msglog_v, h_v, stage_v, part_v, sh_v, sem):
    c = lax.axis_index("c")
    s = lax.axis_index("s")
    rep = c * 8 + s // 2
    part = s % 2
    partner = s - 2 * part + 1

    pltpu.sync_copy(preds0_hbm, preds_v.at[pl.ds(0, N * HID)])
    pltpu.sync_copy(adj_hbm, adj_v)
    pltpu.sync_copy(wns_hbm, wns_v)
    pltpu.sync_copy(wmsg_hbm, wmsg_v)
    pltpu.sync_copy(bns_hbm, bns_v)
    pltpu.sync_copy(bmsg_hbm, bmsg_v)
    pltpu.sync_copy(fmsg_hbm, msglog_v.at[pl.ds(MAXM * MSG, MSG)])

    iotaf = lax.iota(jnp.int32, 16).astype(jnp.float32)
    qnode_v[pl.ds(0, 16)] = jnp.full((16,), rep, jnp.int32)
    qowner_v[pl.ds(0, 16)] = jnp.full((16,), MAXM, jnp.int32)

    hoff = part * (HID // 2)
    pscale = jnp.where(part == 0, 1.0, 0.0)

    def step(t, carry):
        head, tail = carry
        act = head < tail
        node = jnp.clip(qnode_v[pl.ds(head, 16)][0], 0, N - 1)
        owner = jnp.clip(qowner_v[pl.ds(head, 16)][0], 0, MAXM)
        msgch = [msglog_v[pl.ds(owner * MSG + j * 16, 16)]
                 for j in range(MSG // 16)]
        for j in range(MSG // 16):
            h_v[pl.ds(HID + j * 16, 16)] = msgch[j]
        for j in range(HID // 16):
            h_v[pl.ds(j * 16, 16)] = preds_v[pl.ds(node * HID + j * 16, 16)]

        def dot_ns(kc, accs):
            hv = h_v[pl.ds(kc * 16, 16)]
            for l in range(16):
                hk = hv[l]
                base = (kc * 16 + l) * HID + hoff
                accs = tuple(accs[j] + hk * wns_v[pl.ds(base + j * 16, 16)]
                             for j in range(4))
            return accs

        accs = lax.fori_loop(
            0, CAT // 16, dot_ns,
            tuple(bns_v[pl.ds(hoff + j * 16, 16)] for j in range(4)))
        rel = [jnp.maximum(a, 0.0) for a in accs]

        for j in range(4):
            h_v[pl.ds(CAT + j * 16, 16)] = rel[j]
        for m in range(2):
            h_v[pl.ds(CAT + 64 + m * 16, 16)] = (
                msgch[m] * pscale + msgch[2 + m] * (1.0 - pscale))

        def dot_msg(kc, maccs):
            hv = h_v[pl.ds(CAT + kc * 16, 16)]
            kbase = jnp.where(kc < 4, hoff + kc * 16,
                              HID + part * 32 + (kc - 4) * 16)
            for l in range(16):
                hk = hv[l]
                base = (kbase + l) * MSG
                maccs = tuple(maccs[j] + hk * wmsg_v[pl.ds(base + j * 16, 16)]
                              for j in range(4))
            return maccs

        maccs = lax.fori_loop(
            0, 6, dot_msg,
            tuple(bmsg_v[pl.ds(j * 16, 16)] * pscale for j in range(4)))

        for j in range(4):
            stage_v[pl.ds(j * 16, 16)] = rel[j]
            stage_v[pl.ds(64 + j * 16, 16)] = maccs[j]
        par = t % 2
        cp = pltpu.async_copy(stage_v, sh_v.at[par, s], sem)

        arow = adj_v[pl.ds(node * 16, 16)]
        aeff = arow * jnp.where(act, 1.0, 0.0)
        qf = jnp.zeros((16,), jnp.float32)
        pf = jnp.float32(0.0)
        for l in range(16):
            al = aeff[l]
            oh = jnp.where(iotaf == pf, 1.0, 0.0)
            qf = qf + (al * float(l)) * oh
            pf = pf + al
        base = jnp.minimum(tail, MAXM)
        qnode_v[pl.ds(base, 16)] = qf.astype(jnp.int32)
        qowner_v[pl.ds(base, 16)] = jnp.full((16,), t, jnp.int32)

        cp.wait()
        plsc.subcore_barrier()
        pltpu.sync_copy(sh_v.at[par, partner], part_v)

        node_eff = jnp.where(act, node, N)
        poff = (HID // 2) - hoff
        for j in range(4):
            preds_v[pl.ds(node_eff * HID + hoff + j * 16, 16)] = rel[j]
            preds_v[pl.ds(node_eff * HID + poff + j * 16, 16)] = \
                part_v[pl.ds(j * 16, 16)]
        for j in range(4):
            msglog_v[pl.ds(t * MSG + j * 16, 16)] = (
                maccs[j] + part_v[pl.ds(64 + j * 16, 16)])

        return (head + jnp.where(act, 1, 0),
                tail + pf.astype(jnp.int32))

    lax.fori_loop(0, MAXM, step, (jnp.int32(0), jnp.int32(1)))

    @pl.when(part == 0)
    def _():
        pltpu.sync_copy(preds_v.at[pl.ds(rep * HID, HID)],
                        out_hbm.at[pl.ds(rep * HID, HID)])


def _dec_body(p_ref, wdec_ref, bdec_ref, o_ref):
    logits = lax.dot_general(
        p_ref[:], wdec_ref[:], (((1,), (1,)), ((), ())),
        preferred_element_type=jnp.float32) + bdec_ref[:]
    z = logits - jnp.max(logits, axis=-1, keepdims=True)
    o_ref[:] = z - jnp.log(jnp.sum(jnp.exp(z), axis=-1, keepdims=True))


def kernel(x, edge_index, W_enc, b_enc, W_ns, b_ns, W_msg, b_msg,
           W_dec, b_dec, first_message):
    ei = edge_index.astype(jnp.int32)
    ei = jnp.concatenate([ei, ei, ei, ei], axis=0)

    preds0, adj, wnst, wmsgt = pl.pallas_call(
        _enc_body,
        out_shape=[
            jax.ShapeDtypeStruct((N, HID), jnp.float32),
            jax.ShapeDtypeStruct((N, N), jnp.float32),
            jax.ShapeDtypeStruct((CAT, HID), jnp.float32),
            jax.ShapeDtypeStruct((CAT, MSG), jnp.float32),
        ],
    )(x, W_enc, b_enc.reshape(1, HID), ei, W_ns, W_msg)

    mesh = plsc.VectorSubcoreMesh(core_axis_name="c", subcore_axis_name="s")
    walk = pl.kernel(
        _walk_body, mesh=mesh,
        out_type=jax.ShapeDtypeStruct((N * HID,), jnp.float32),
        scratch_types=[
            pltpu.VMEM(((N + 1) * HID,), jnp.float32),
            pltpu.VMEM((N * N,), jnp.float32),
            pltpu.VMEM((CAT * HID,), jnp.float32),
            pltpu.VMEM((CAT * MSG,), jnp.float32),
            pltpu.VMEM((HID,), jnp.float32),
            pltpu.VMEM((MSG,), jnp.float32),
            pltpu.VMEM((QPAD,), jnp.int32),
            pltpu.VMEM((QPAD,), jnp.int32),
            pltpu.VMEM(((MAXM + 1) * MSG,), jnp.float32),
            pltpu.VMEM((CAT + 96,), jnp.float32),
            pltpu.VMEM((HID,), jnp.float32),
            pltpu.VMEM((HID,), jnp.float32),
            pltpu.VMEM_SHARED((2, 16, HID), jnp.float32),
            pltpu.SemaphoreType.DMA,
        ],
    )
    finalp = walk(preds0.reshape(-1), adj.reshape(-1), wnst.reshape(-1),
                  wmsgt.reshape(-1), b_ns, b_msg, first_message.reshape(-1))

    out = pl.pallas_call(
        _dec_body,
        out_shape=jax.ShapeDtypeStruct((N, OUT_F), jnp.float32),
    )(finalp.reshape(N, HID), W_dec, b_dec.reshape(1, OUT_F))
    return (out, 0)
